# baseline (speedup 1.0000x reference)
import functools

import jax
import jax.numpy as jnp
from jax import lax
from jax.experimental import pallas as pl
from jax.experimental.pallas import tpu as pltpu
from jax.experimental.pallas import tpu_sc as plsc

V = 1000000
D = 32
NW = 32
BLK = 512
NBLK = 1953
CHUNK = 512
VPAD = 34
WSUP_LINES = V * VPAD // 128
ABO_LINES = BLK * VPAD // 128
TAIL_LINES = 64 * VPAD // 128
NF = 26
NB = 16384


def _iota16():
    return lax.iota(jnp.int32, 16)


def _splat(v):
    return jnp.full((16,), v, jnp.int32)


@functools.lru_cache(maxsize=None)
def _make_transpose_kernel():
    mesh = plsc.VectorSubcoreMesh(core_axis_name="c", subcore_axis_name="s")

    @functools.partial(
        pl.kernel,
        mesh=mesh,
        compiler_params=pltpu.CompilerParams(use_tc_tiling_on_sc=True, needs_layout_passes=False, disable_bounds_checks=True),
        out_type=jax.ShapeDtypeStruct((WSUP_LINES, 128), jnp.float32),
        scratch_types=[
            pltpu.VMEM((32, BLK), jnp.float32),
            pltpu.VMEM((32, BLK), jnp.float32),
            pltpu.VMEM((ABO_LINES, 128), jnp.float32),
            pltpu.VMEM((ABO_LINES, 128), jnp.float32),
            pltpu.SemaphoreType.DMA,
            pltpu.SemaphoreType.DMA,
            pltpu.SemaphoreType.DMA,
            pltpu.SemaphoreType.DMA,
        ],
    )
    def k(wt_hbm, wtail_hbm, wsup_hbm, bin0, bin1, bout0, bout1,
          gi0, gi1, so0, so1):
        wid = lax.axis_index("s") * 2 + lax.axis_index("c")
        bins = (bin0, bin1)
        bouts = (bout0, bout1)
        gis = (gi0, gi1)
        sos = (so0, so1)
        niter = 62

        def in_copy(blk, b):
            return pltpu.make_async_copy(
                wt_hbm.at[:, pl.ds(blk * BLK, BLK)], bins[b], gis[b]
            )

        def out_copy(blk, b):
            return pltpu.make_async_copy(
                bouts[b], wsup_hbm.at[pl.ds(blk * ABO_LINES, ABO_LINES)],
                sos[b]
            )

        @pl.when(wid < NBLK)
        def _():
            in_copy(wid, 0).start()

        def body(outer, carry):
            for sub in range(2):
                it = outer * 2 + sub
                blk = wid + NW * it
                nblk = blk + NW
                valid = blk < NBLK

                @pl.when(valid)
                def _():
                    in_copy(blk, sub).wait()

                    @pl.when(nblk < NBLK)
                    def _():
                        in_copy(nblk, 1 - sub).start()

                    @pl.when(it >= 2)
                    def _():
                        out_copy(blk - 2 * NW, sub).wait()

                    bi, bo = bins[sub], bouts[sub]
                    v34 = _iota16() * VPAD

                    @plsc.parallel_loop(0, 1024, 1, unroll=8)
                    def trow(z):
                        i0 = (z >> 5) * 16
                        c = z & 31
                        v = bi[c, pl.ds(i0, 16)]
                        flat = v34 + (VPAD * i0 + c)
                        plsc.store_scatter(bo, [flat >> 7, flat & 127], v)

                    out_copy(blk, sub).start()

            return carry

        lax.fori_loop(0, niter // 2, body, 0)

        out_copy(wid, 0).wait()
        out_copy(wid, 1).wait()

        @pl.when(wid == 0)
        def _():
            pltpu.sync_copy(wtail_hbm, bin0.at[:, pl.ds(0, 128)])

            v34t = _iota16() * VPAD

            def trow_t(z, c2):
                i0 = (z >> 5) * 16
                c = z & 31
                v = bin0[c, pl.ds(i0, 16)]
                flat = v34t + (VPAD * i0 + c)
                plsc.store_scatter(bout0, [flat >> 7, flat & 127], v)
                return c2

            lax.fori_loop(0, 128, trow_t, 0)
            pltpu.sync_copy(bout0.at[pl.ds(0, TAIL_LINES)],
                            wsup_hbm.at[pl.ds(NBLK * ABO_LINES, TAIL_LINES)])

    return k


@functools.lru_cache(maxsize=None)
def _make_gather_kernel():
    nch = (NF * NB) // (NW * CHUNK)
    mesh = plsc.VectorSubcoreMesh(core_axis_name="c", subcore_axis_name="s")

    @functools.partial(
        pl.kernel,
        mesh=mesh,
        compiler_params=pltpu.CompilerParams(use_tc_tiling_on_sc=False, needs_layout_passes=False, disable_bounds_checks=True),
        out_type=jax.ShapeDtypeStruct((NF, 4, NB // 128, 8, 128), jnp.float32),
        scratch_types=[
            pltpu.VMEM((nch, CHUNK), jnp.int32),
            pltpu.VMEM((CHUNK, VPAD), jnp.float32),
            pltpu.VMEM((CHUNK, VPAD), jnp.float32),
            pltpu.VMEM((4, 4, 8, 128), jnp.float32),
            pltpu.VMEM((4, 4, 8, 128), jnp.float32),
            pltpu.SemaphoreType.DMA,
            pltpu.SemaphoreType.DMA,
            pltpu.SemaphoreType.DMA,
            pltpu.SemaphoreType.DMA,
        ],
    )
    def k(xq_hbm, w_hbm, p_hbm, idx_v, r0, r1, o0, o1, g0, g1, s0, s1):
        wid = lax.axis_index("s") * 2 + lax.axis_index("c")
        rows = (r0, r1)
        obufs = (o0, o1)
        gs = (g0, g1)
        ss = (s0, s1)

        pltpu.sync_copy(xq_hbm.at[wid], idx_v)

        def gth(j, b):
            return pltpu.make_async_copy(
                w_hbm.at[idx_v.at[j]], rows[b], gs[b]
            )

        def pout(j, b):
            g = wid * nch + j
            return pltpu.make_async_copy(
                obufs[b],
                p_hbm.at[g // 32, :, pl.ds((g % 32) * 4, 4)],
                ss[b],
            )

        gth(0, 0).start()
        gth(1, 1).start()

        def body(i, carry):
            for sub in range(2):
                j = i * 2 + sub
                gth(j, sub).wait()

                @pl.when(j >= 2)
                def _():
                    pout(j - 2, sub).wait()

                ri, ob = rows[sub], obufs[sub]
                rows16 = tuple(_iota16() + 16 * kk for kk in range(8))

                @plsc.parallel_loop(0, 128, 1, unroll=4)
                def ttile(z):
                    t = z >> 5
                    u = (z >> 3) & 3
                    r = z & 7
                    col = _splat(((z >> 5) << 3) | (z & 7))
                    base = _splat(u << 7)
                    for kk in range(8):
                        v = plsc.load_gather(ri, [rows16[kk] + base, col])
                        ob[t, u, r, pl.ds(16 * kk, 16)] = v

                pout(j, sub).start()

                @pl.when(j + 2 < nch)
                def _():
                    gth(j + 2, sub).start()

            return carry

        lax.fori_loop(0, nch // 2, body, 0)
        pout(nch - 2, 0).wait()
        pout(nch - 1, 1).wait()

    return k


def kernel(x, weight):
    wt = weight.T
    wtail = jnp.pad(wt[:, NBLK * BLK:], ((0, 0), (0, 64)))
    w_sup = _make_transpose_kernel()(wt, wtail)
    w_lin = w_sup.reshape(V, VPAD)
    xq = x.T.astype(jnp.int32).reshape(NW, (NF * NB) // (NW * CHUNK), CHUNK)
    p = _make_gather_kernel()(xq, w_lin)
    out = p.transpose(0, 1, 3, 2, 4).reshape(NF, D, NB).transpose(2, 0, 1)
    return out


# speedup vs baseline: 2.6595x; 2.6595x over previous
import functools

import jax
import jax.numpy as jnp
from jax import lax
from jax.experimental import pallas as pl
from jax.experimental.pallas import tpu as pltpu
from jax.experimental.pallas import tpu_sc as plsc

V = 1000000
D = 32
NW = 32
BLK = 512
NBLK = 1953
CHUNK = 512
NF = 26
NB = 16384


def _iota16():
    return lax.iota(jnp.int32, 16)


def _splat(v):
    return jnp.full((16,), v, jnp.int32)


@functools.lru_cache(maxsize=None)
def _make_transpose_kernel():
    mesh = plsc.VectorSubcoreMesh(core_axis_name="c", subcore_axis_name="s")

    @functools.partial(
        pl.kernel,
        mesh=mesh,
        compiler_params=pltpu.CompilerParams(use_tc_tiling_on_sc=True, needs_layout_passes=False, disable_bounds_checks=True),
        out_type=jax.ShapeDtypeStruct((V // 4, 128), jnp.float32),
        scratch_types=[
            pltpu.VMEM((32, BLK), jnp.float32),
            pltpu.VMEM((32, BLK), jnp.float32),
            pltpu.VMEM((128, 128), jnp.float32),
            pltpu.VMEM((128, 128), jnp.float32),
            pltpu.SemaphoreType.DMA,
            pltpu.SemaphoreType.DMA,
            pltpu.SemaphoreType.DMA,
            pltpu.SemaphoreType.DMA,
        ],
    )
    def k(wt_hbm, wtail_hbm, wsup_hbm, bin0, bin1, bout0, bout1,
          gi0, gi1, so0, so1):
        wid = lax.axis_index("s") * 2 + lax.axis_index("c")
        bins = (bin0, bin1)
        bouts = (bout0, bout1)
        gis = (gi0, gi1)
        sos = (so0, so1)
        niter = 62

        def in_copy(blk, b):
            return pltpu.make_async_copy(
                wt_hbm.at[:, pl.ds(blk * BLK, BLK)], bins[b], gis[b]
            )

        def out_copy(blk, b):
            return pltpu.make_async_copy(
                bouts[b], wsup_hbm.at[pl.ds(blk * 128, 128)], sos[b]
            )

        @pl.when(wid < NBLK)
        def _():
            in_copy(wid, 0).start()

        def body(outer, carry):
            for sub in range(2):
                it = outer * 2 + sub
                blk = wid + NW * it
                nblk = blk + NW
                valid = blk < NBLK

                @pl.when(valid)
                def _():
                    in_copy(blk, sub).wait()

                    @pl.when(nblk < NBLK)
                    def _():
                        in_copy(nblk, 1 - sub).start()

                    @pl.when(it >= 2)
                    def _():
                        out_copy(blk - 2 * NW, sub).wait()

                    bi, bo = bins[sub], bouts[sub]
                    rows16 = (_iota16(), _iota16() + 16)

                    @plsc.parallel_loop(0, BLK, 1, unroll=16)
                    def trow(fl):
                        col = _splat(fl)
                        line = fl >> 2
                        o = (fl & 3) * 32
                        for h in range(2):
                            v = plsc.load_gather(bi, [rows16[h], col])
                            bo[line, pl.ds(o + 16 * h, 16)] = v

                    out_copy(blk, sub).start()

            return carry

        lax.fori_loop(0, niter // 2, body, 0)

        out_copy(wid, 0).wait()
        out_copy(wid, 1).wait()

        @pl.when(wid == 0)
        def _():
            pltpu.sync_copy(wtail_hbm, bin0.at[:, pl.ds(0, 128)])

            def trow_t(line, c2):
                for q in range(4):
                    for h in range(2):
                        v = plsc.load_gather(
                            bin0, [_iota16() + 16 * h, _splat(4 * line + q)]
                        )
                        bout0[line, pl.ds(32 * q + 16 * h, 16)] = v
                return c2

            lax.fori_loop(0, 16, trow_t, 0)
            pltpu.sync_copy(bout0.at[pl.ds(0, 16)],
                            wsup_hbm.at[pl.ds(NBLK * 128, 16)])

    return k


@functools.lru_cache(maxsize=None)
def _make_gather_kernel():
    nch = (NF * NB) // (NW * CHUNK)
    mesh = plsc.VectorSubcoreMesh(core_axis_name="c", subcore_axis_name="s")

    @functools.partial(
        pl.kernel,
        mesh=mesh,
        compiler_params=pltpu.CompilerParams(use_tc_tiling_on_sc=False, needs_layout_passes=False, disable_bounds_checks=True),
        out_type=jax.ShapeDtypeStruct((NF, 4, NB // 128, 8, 128), jnp.float32),
        scratch_types=[
            pltpu.VMEM((nch, CHUNK), jnp.int32),
            pltpu.VMEM((CHUNK, D), jnp.float32),
            pltpu.VMEM((CHUNK, D), jnp.float32),
            pltpu.VMEM((4, 4, 8, 128), jnp.float32),
            pltpu.VMEM((4, 4, 8, 128), jnp.float32),
            pltpu.SemaphoreType.DMA,
            pltpu.SemaphoreType.DMA,
            pltpu.SemaphoreType.DMA,
            pltpu.SemaphoreType.DMA,
        ],
    )
    def k(xq_hbm, w_hbm, p_hbm, idx_v, r0, r1, o0, o1, g0, g1, s0, s1):
        wid = lax.axis_index("s") * 2 + lax.axis_index("c")
        rows = (r0, r1)
        obufs = (o0, o1)
        gs = (g0, g1)
        ss = (s0, s1)

        pltpu.sync_copy(xq_hbm.at[wid], idx_v)

        def gth(j, b):
            return pltpu.make_async_copy(
                w_hbm.at[idx_v.at[j]], rows[b], gs[b]
            )

        def pout(j, b):
            g = wid * nch + j
            return pltpu.make_async_copy(
                obufs[b],
                p_hbm.at[g // 32, :, pl.ds((g % 32) * 4, 4)],
                ss[b],
            )

        gth(0, 0).start()
        gth(1, 1).start()

        def body(i, carry):
            for sub in range(2):
                j = i * 2 + sub
                gth(j, sub).wait()

                @pl.when(j >= 2)
                def _():
                    pout(j - 2, sub).wait()

                ri, ob = rows[sub], obufs[sub]
                rows16 = tuple(_iota16() + 16 * kk for kk in range(8))

                @plsc.parallel_loop(0, 128, 1, unroll=8)
                def ttile(z):
                    t = z >> 5
                    u = (z >> 3) & 3
                    r = z & 7
                    col = _splat(((z >> 5) << 3) | (z & 7))
                    base = _splat(u << 7)
                    for kk in range(8):
                        v = plsc.load_gather(ri, [rows16[kk] + base, col])
                        ob[t, u, r, pl.ds(16 * kk, 16)] = v

                pout(j, sub).start()

                @pl.when(j + 2 < nch)
                def _():
                    gth(j + 2, sub).start()

            return carry

        lax.fori_loop(0, nch // 2, body, 0)
        pout(nch - 2, 0).wait()
        pout(nch - 1, 1).wait()

    return k


def kernel(x, weight):
    wt = weight.T
    wtail = jnp.pad(wt[:, NBLK * BLK:], ((0, 0), (0, 64)))
    w_sup = _make_transpose_kernel()(wt, wtail)
    w_lin = w_sup.reshape(V, D)
    xq = x.T.astype(jnp.int32).reshape(NW, (NF * NB) // (NW * CHUNK), CHUNK)
    p = _make_gather_kernel()(xq, w_lin)
    out = p.transpose(0, 1, 3, 2, 4).reshape(NF, D, NB).transpose(2, 0, 1)
    return out


# speedup vs baseline: 2.6635x; 1.0015x over previous
import functools

import jax
import jax.numpy as jnp
from jax import lax
from jax.experimental import pallas as pl
from jax.experimental.pallas import tpu as pltpu
from jax.experimental.pallas import tpu_sc as plsc

V = 1000000
D = 32
NW = 32
BLK = 512
NBLK = 1953
CHUNK = 512
NF = 26
NB = 16384


def _iota16():
    return lax.iota(jnp.int32, 16)


def _splat(v):
    return jnp.full((16,), v, jnp.int32)


@functools.lru_cache(maxsize=None)
def _make_transpose_kernel():
    mesh = plsc.VectorSubcoreMesh(core_axis_name="c", subcore_axis_name="s")

    @functools.partial(
        pl.kernel,
        mesh=mesh,
        compiler_params=pltpu.CompilerParams(use_tc_tiling_on_sc=True, needs_layout_passes=False, disable_bounds_checks=True),
        out_type=jax.ShapeDtypeStruct((V // 4, 128), jnp.float32),
        scratch_types=[
            pltpu.VMEM((32, BLK), jnp.float32),
            pltpu.VMEM((32, BLK), jnp.float32),
            pltpu.VMEM((128, 128), jnp.float32),
            pltpu.VMEM((128, 128), jnp.float32),
            pltpu.SemaphoreType.DMA,
            pltpu.SemaphoreType.DMA,
            pltpu.SemaphoreType.DMA,
            pltpu.SemaphoreType.DMA,
        ],
    )
    def k(wt_hbm, wtail_hbm, wsup_hbm, bin0, bin1, bout0, bout1,
          gi0, gi1, so0, so1):
        wid = lax.axis_index("s") * 2 + lax.axis_index("c")
        bins = (bin0, bin1)
        bouts = (bout0, bout1)
        gis = (gi0, gi1)
        sos = (so0, so1)
        niter = 62

        def in_copy(blk, b):
            return pltpu.make_async_copy(
                wt_hbm.at[:, pl.ds(blk * BLK, BLK)], bins[b], gis[b]
            )

        def out_copy(blk, b):
            return pltpu.make_async_copy(
                bouts[b], wsup_hbm.at[pl.ds(blk * 128, 128)], sos[b]
            )

        @pl.when(wid < NBLK)
        def _():
            in_copy(wid, 0).start()

        def body(outer, carry):
            for sub in range(2):
                it = outer * 2 + sub
                blk = wid + NW * it
                nblk = blk + NW
                valid = blk < NBLK

                @pl.when(valid)
                def _():
                    in_copy(blk, sub).wait()

                    @pl.when(nblk < NBLK)
                    def _():
                        in_copy(nblk, 1 - sub).start()

                    @pl.when(it >= 2)
                    def _():
                        out_copy(blk - 2 * NW, sub).wait()

                    bi, bo = bins[sub], bouts[sub]
                    rows16 = (_iota16(), _iota16() + 16)

                    @plsc.parallel_loop(0, BLK, 1, unroll=8)
                    def trow(fl):
                        col = _splat(fl)
                        line = fl >> 2
                        o = (fl & 3) * 32
                        for h in range(2):
                            v = plsc.load_gather(bi, [rows16[h], col])
                            bo[line, pl.ds(o + 16 * h, 16)] = v

                    out_copy(blk, sub).start()

            return carry

        lax.fori_loop(0, niter // 2, body, 0)

        out_copy(wid, 0).wait()
        out_copy(wid, 1).wait()

        @pl.when(wid == 0)
        def _():
            pltpu.sync_copy(wtail_hbm, bin0.at[:, pl.ds(0, 128)])

            def trow_t(line, c2):
                for q in range(4):
                    for h in range(2):
                        v = plsc.load_gather(
                            bin0, [_iota16() + 16 * h, _splat(4 * line + q)]
                        )
                        bout0[line, pl.ds(32 * q + 16 * h, 16)] = v
                return c2

            lax.fori_loop(0, 16, trow_t, 0)
            pltpu.sync_copy(bout0.at[pl.ds(0, 16)],
                            wsup_hbm.at[pl.ds(NBLK * 128, 16)])

    return k


@functools.lru_cache(maxsize=None)
def _make_gather_kernel():
    nch = (NF * NB) // (NW * CHUNK)
    mesh = plsc.VectorSubcoreMesh(core_axis_name="c", subcore_axis_name="s")

    @functools.partial(
        pl.kernel,
        mesh=mesh,
        compiler_params=pltpu.CompilerParams(use_tc_tiling_on_sc=False, needs_layout_passes=False, disable_bounds_checks=True),
        out_type=jax.ShapeDtypeStruct((NF, 4, NB // 128, 8, 128), jnp.float32),
        scratch_types=[
            pltpu.VMEM((nch, CHUNK), jnp.int32),
            pltpu.VMEM((CHUNK, D), jnp.float32),
            pltpu.VMEM((CHUNK, D), jnp.float32),
            pltpu.VMEM((4, 4, 8, 128), jnp.float32),
            pltpu.VMEM((4, 4, 8, 128), jnp.float32),
            pltpu.SemaphoreType.DMA,
            pltpu.SemaphoreType.DMA,
            pltpu.SemaphoreType.DMA,
            pltpu.SemaphoreType.DMA,
        ],
    )
    def k(xq_hbm, w_hbm, p_hbm, idx_v, r0, r1, o0, o1, g0, g1, s0, s1):
        wid = lax.axis_index("s") * 2 + lax.axis_index("c")
        rows = (r0, r1)
        obufs = (o0, o1)
        gs = (g0, g1)
        ss = (s0, s1)

        pltpu.sync_copy(xq_hbm.at[wid], idx_v)

        def gth(j, b):
            return pltpu.make_async_copy(
                w_hbm.at[idx_v.at[j]], rows[b], gs[b]
            )

        def pout(j, b):
            g = wid * nch + j
            return pltpu.make_async_copy(
                obufs[b],
                p_hbm.at[g // 32, :, pl.ds((g % 32) * 4, 4)],
                ss[b],
            )

        gth(0, 0).start()
        gth(1, 1).start()

        def body(i, carry):
            for sub in range(2):
                j = i * 2 + sub
                gth(j, sub).wait()

                @pl.when(j >= 2)
                def _():
                    pout(j - 2, sub).wait()

                ri, ob = rows[sub], obufs[sub]
                rows16 = tuple(_iota16() + 16 * kk for kk in range(8))

                @plsc.parallel_loop(0, 128, 1, unroll=4)
                def ttile(z):
                    t = z >> 5
                    u = (z >> 3) & 3
                    r = z & 7
                    col = _splat(((z >> 5) << 3) | (z & 7))
                    base = _splat(u << 7)
                    for kk in range(8):
                        v = plsc.load_gather(ri, [rows16[kk] + base, col])
                        ob[t, u, r, pl.ds(16 * kk, 16)] = v

                pout(j, sub).start()

                @pl.when(j + 2 < nch)
                def _():
                    gth(j + 2, sub).start()

            return carry

        lax.fori_loop(0, nch // 2, body, 0)
        pout(nch - 2, 0).wait()
        pout(nch - 1, 1).wait()

    return k


def kernel(x, weight):
    wt = weight.T
    wtail = jnp.pad(wt[:, NBLK * BLK:], ((0, 0), (0, 64)))
    w_sup = _make_transpose_kernel()(wt, wtail)
    w_lin = w_sup.reshape(V, D)
    xq = x.T.astype(jnp.int32).reshape(NW, (NF * NB) // (NW * CHUNK), CHUNK)
    p = _make_gather_kernel()(xq, w_lin)
    out = p.transpose(0, 1, 3, 2, 4).reshape(NF, D, NB).transpose(2, 0, 1)
    return out
